# baseline (speedup 1.0000x reference)
import jax
import jax.numpy as jnp
from jax import lax
from jax.experimental import pallas as pl
from jax.experimental.pallas import tpu as pltpu
from jax.experimental.pallas import tpu_sc as plsc

K_FRAC = 0.1

NC, NS, L = 2, 16, 16
NW = NC * NS

ROWS = 16384
D = 2048
ROWS_SC = 6144
ROWS_PER_W = ROWS_SC // NW
GROUPS = ROWS_PER_W // L

IMIN = -2**31


def _sc_body(x_hbm, o_hbm, xb0, xb1, hist, isem0, isem1, osem0, osem1):
    k = max(1, int(D * K_FRAC))
    wid = lax.axis_index("s") * NC + lax.axis_index("c")
    lane = lax.iota(jnp.int32, L)
    ones = jnp.ones((L,), jnp.int32)
    zeros = jnp.zeros((L,), jnp.int32)
    kv = jnp.full((L,), k, jnp.int32)
    lane_base = lane * D
    row0 = wid * ROWS_PER_W

    bufs = (xb0, xb1)
    isems = (isem0, isem1)
    osems = (osem0, osem1)

    def start_in(g, b):
        return pltpu.async_copy(
            x_hbm.at[pl.ds((row0 + g * L), L), :], bufs[b], isems[b])

    def start_out(g, b):
        return pltpu.async_copy(
            bufs[b], o_hbm.at[pl.ds((row0 + g * L), L), :], osems[b])

    @plsc.parallel_loop(0, 512, unroll=8)
    def _z(i):
        hist[pl.ds(i * L, L)] = zeros

    def select(kcur):
        @plsc.parallel_loop(0, 512, unroll=8, carry=(zeros, zeros, zeros))
        def res(i, carry):
            acc, chosen, cntgt = carry
            d = 511 - i
            h = hist[pl.ds(d * L, L)]
            hist[pl.ds(d * L, L)] = zeros
            acc2 = acc + h
            newly = (acc < kcur) & (acc2 >= kcur)
            dv = jnp.full((L,), d, jnp.int32)
            chosen = jnp.where(newly, dv, chosen)
            cntgt = jnp.where(newly, acc, cntgt)
            return acc2, chosen, cntgt
        acc, chosen, cntgt = res
        return chosen, kcur - cntgt

    def compute(g, b, mid):
        xbuf = bufs[b]

        @plsc.parallel_loop(0, D, unroll=8)
        def _p1(j):
            col = (j + lane) & (D - 1)
            bits = plsc.bitcast(
                plsc.load_gather(xbuf, [lane, col]), jnp.int32)
            u = bits ^ (lax.shift_right_arithmetic(bits, 31) | IMIN)
            plsc.store_scatter(xbuf, [lane, col],
                               plsc.bitcast(u, jnp.float32))
            addr = (lax.shift_right_logical(u, 19) & 0x1FF0) | lane
            plsc.addupdate_scatter(hist, [addr], ones)

        mid()

        d1, k2 = select(kv)

        @plsc.parallel_loop(0, D, unroll=8)
        def _p2(j):
            col = (j + lane) & (D - 1)
            u = plsc.bitcast(plsc.load_gather(xbuf, [lane, col]), jnp.int32)
            match = lax.shift_right_logical(u, 23) == d1
            addr = (lax.shift_right_logical(u, 10) & 0x1FF0) | lane
            plsc.addupdate_scatter(hist, [addr], ones, mask=match)

        d2, _ = select(k2)

        t_u = ((d1 << 9) | d2) << 14
        tbits = t_u ^ (~lax.shift_right_arithmetic(t_u, 31) | IMIN)
        t = plsc.bitcast(tbits, jnp.float32)

        @plsc.parallel_loop(0, D, unroll=8)
        def _ap(j):
            col = (j + lane) & (D - 1)
            u = plsc.bitcast(plsc.load_gather(xbuf, [lane, col]), jnp.int32)
            xb = u ^ (~lax.shift_right_arithmetic(u, 31) | IMIN)
            v = plsc.bitcast(xb, jnp.float32)
            y = v / (1.0 + jnp.exp(t - v))
            plsc.store_scatter(xbuf, [lane, col], y)

    def wait_in(g, b):
        pltpu.make_async_copy(
            x_hbm.at[pl.ds((row0 + g * L), L), :], bufs[b],
            isems[b]).wait()

    def wait_out(g, b):
        pltpu.make_async_copy(
            bufs[b], o_hbm.at[pl.ds((row0 + g * L), L), :],
            osems[b]).wait()

    start_in(0, 0)
    start_in(1, 1)

    def pair(h, c):
        wait_in(2 * h, 0)

        def mid0():
            @pl.when(h > 0)
            def _():
                wait_out(2 * h - 1, 1)
                start_in(2 * h + 1, 1)

        compute(2 * h, 0, mid0)
        start_out(2 * h, 0)

        def mid1():
            @pl.when(h + 1 < GROUPS // 2)
            def _():
                wait_out(2 * h, 0)
                start_in(2 * h + 2, 0)

        wait_in(2 * h + 1, 1)
        compute(2 * h + 1, 1, mid1)
        start_out(2 * h + 1, 1)

        @pl.when(h + 1 >= GROUPS // 2)
        def _():
            wait_out(2 * h, 0)
        return c

    lax.fori_loop(0, GROUPS // 2, pair, 0)
    wait_out(GROUPS - 1, 1)


def _tc_body(x_ref, o_ref):
    x = x_ref[...]
    rows, d = x.shape
    k = max(1, int(d * K_FRAC))
    u = pltpu.bitcast(x, jnp.uint32)
    neg = (u >> 31) == 1
    u = jnp.where(neg, ~u, u | jnp.uint32(0x80000000))
    ones_col = jnp.ones((d, 8), jnp.bfloat16)
    prefix = jnp.zeros((rows, 1), jnp.uint32)
    for bit in range(31, 15, -1):
        cand = prefix | jnp.uint32(1 << bit)
        mask = (u >= cand).astype(jnp.bfloat16)
        cnt = jax.lax.dot_general(
            mask, ones_col, (((1,), (0,)), ((), ())),
            preferred_element_type=jnp.float32)[:, :1]
        prefix = jnp.where(cnt >= float(k), cand, prefix)
    tneg = (prefix >> 31) == 0
    tbits = jnp.where(tneg, ~prefix, prefix & jnp.uint32(0x7FFFFFFF))
    t = pltpu.bitcast(tbits, jnp.float32)
    o_ref[...] = x * (1.0 / (1.0 + jnp.exp(t - x)))


def _tc_call(xf):
    rows_blk = 256
    rows = ROWS - ROWS_SC
    off = ROWS_SC // rows_blk
    return pl.pallas_call(
        _tc_body,
        grid=(rows // rows_blk,),
        in_specs=[pl.BlockSpec((rows_blk, D), lambda i: (i + off, 0))],
        out_specs=pl.BlockSpec((rows_blk, D), lambda i: (i + off, 0)),
        out_shape=jax.ShapeDtypeStruct((ROWS, D), jnp.float32),
    )(xf)


def kernel(x):
    b, s, d = x.shape
    xf = x.reshape(b * s, d)
    mesh = plsc.VectorSubcoreMesh(
        core_axis_name="c", subcore_axis_name="s",
        num_cores=NC, num_subcores=NS)
    fn = pl.kernel(
        _sc_body,
        out_type=jax.ShapeDtypeStruct((ROWS_SC, D), jnp.float32),
        mesh=mesh,
        compiler_params=pltpu.CompilerParams(needs_layout_passes=False),
        scratch_types=[
            pltpu.VMEM((L, D), jnp.float32),
            pltpu.VMEM((L, D), jnp.float32),
            pltpu.VMEM((512 * L,), jnp.int32),
            pltpu.SemaphoreType.DMA,
            pltpu.SemaphoreType.DMA,
            pltpu.SemaphoreType.DMA,
            pltpu.SemaphoreType.DMA,
        ],
    )
    sc_out = fn(xf)
    tc_out = _tc_call(xf)
    out = lax.dynamic_update_slice(tc_out, sc_out, (0, 0))
    return out.reshape(b, s, d)


# speedup vs baseline: 1.4031x; 1.4031x over previous
import jax
import jax.numpy as jnp
from jax import lax
from jax.experimental import pallas as pl
from jax.experimental.pallas import tpu as pltpu
from jax.experimental.pallas import tpu_sc as plsc

K_FRAC = 0.1

NC, NS, L = 2, 16, 16
NW = NC * NS

ROWS = 16384
D = 2048
ROWS_SC = 6144
ROWS_PER_W = ROWS_SC // NW
GROUPS = ROWS_PER_W // L

IMIN = -2**31


def _sc_body(x_hbm, o_hbm, xb0, xb1, hist, isem0, isem1, osem0, osem1):
    k = max(1, int(D * K_FRAC))
    wid = lax.axis_index("s") * NC + lax.axis_index("c")
    lane = lax.iota(jnp.int32, L)
    ones = jnp.ones((L,), jnp.int32)
    zeros = jnp.zeros((L,), jnp.int32)
    kv = jnp.full((L,), k, jnp.int32)
    lane_base = lane * D
    row0 = wid * ROWS_PER_W

    bufs = (xb0, xb1)
    isems = (isem0, isem1)
    osems = (osem0, osem1)

    def start_in(g, b):
        return pltpu.async_copy(
            x_hbm.at[pl.ds((row0 + g * L), L), :], bufs[b], isems[b])

    def start_out(g, b):
        return pltpu.async_copy(
            bufs[b], o_hbm.at[pl.ds((row0 + g * L), L), :], osems[b])

    @plsc.parallel_loop(0, 512, unroll=8)
    def _z(i):
        hist[pl.ds(i * L, L)] = zeros

    def select(kcur):
        @plsc.parallel_loop(0, 512, unroll=8, carry=(zeros, zeros, zeros))
        def res(i, carry):
            acc, chosen, cntgt = carry
            d = 511 - i
            h = hist[pl.ds(d * L, L)]
            hist[pl.ds(d * L, L)] = zeros
            acc2 = acc + h
            newly = (acc < kcur) & (acc2 >= kcur)
            dv = jnp.full((L,), d, jnp.int32)
            chosen = jnp.where(newly, dv, chosen)
            cntgt = jnp.where(newly, acc, cntgt)
            return acc2, chosen, cntgt
        acc, chosen, cntgt = res
        return chosen, kcur - cntgt

    def compute(g, b, mid):
        xbuf = bufs[b]

        @plsc.parallel_loop(0, D, unroll=8)
        def _p1(j):
            col = (j + lane) & (D - 1)
            bits = plsc.bitcast(
                plsc.load_gather(xbuf, [lane, col]), jnp.int32)
            u = bits ^ (lax.shift_right_arithmetic(bits, 31) | IMIN)
            plsc.store_scatter(xbuf, [lane, col],
                               plsc.bitcast(u, jnp.float32))
            addr = (lax.shift_right_logical(u, 19) & 0x1FF0) | lane
            plsc.addupdate_scatter(hist, [addr], ones)

        mid()

        d1, k2 = select(kv)

        @plsc.parallel_loop(0, D, unroll=8)
        def _p2(j):
            col = (j + lane) & (D - 1)
            u = plsc.bitcast(plsc.load_gather(xbuf, [lane, col]), jnp.int32)
            match = lax.shift_right_logical(u, 23) == d1
            addr = (lax.shift_right_logical(u, 10) & 0x1FF0) | lane
            plsc.addupdate_scatter(hist, [addr], ones, mask=match)

        d2, _ = select(k2)

        t_u = ((d1 << 9) | d2) << 14
        tbits = t_u ^ (~lax.shift_right_arithmetic(t_u, 31) | IMIN)
        t = plsc.bitcast(tbits, jnp.float32)

        @plsc.parallel_loop(0, D, unroll=8)
        def _ap(j):
            col = (j + lane) & (D - 1)
            u = plsc.bitcast(plsc.load_gather(xbuf, [lane, col]), jnp.int32)
            xb = u ^ (~lax.shift_right_arithmetic(u, 31) | IMIN)
            v = plsc.bitcast(xb, jnp.float32)
            y = v / (1.0 + jnp.exp(t - v))
            plsc.store_scatter(xbuf, [lane, col], y)

    def wait_in(g, b):
        pltpu.make_async_copy(
            x_hbm.at[pl.ds((row0 + g * L), L), :], bufs[b],
            isems[b]).wait()

    def wait_out(g, b):
        pltpu.make_async_copy(
            bufs[b], o_hbm.at[pl.ds((row0 + g * L), L), :],
            osems[b]).wait()

    start_in(0, 0)
    start_in(1, 1)

    def pair(h, c):
        wait_in(2 * h, 0)

        def mid0():
            @pl.when(h > 0)
            def _():
                wait_out(2 * h - 1, 1)
                start_in(2 * h + 1, 1)

        compute(2 * h, 0, mid0)
        start_out(2 * h, 0)

        def mid1():
            @pl.when(h + 1 < GROUPS // 2)
            def _():
                wait_out(2 * h, 0)
                start_in(2 * h + 2, 0)

        wait_in(2 * h + 1, 1)
        compute(2 * h + 1, 1, mid1)
        start_out(2 * h + 1, 1)

        @pl.when(h + 1 >= GROUPS // 2)
        def _():
            wait_out(2 * h, 0)
        return c

    lax.fori_loop(0, GROUPS // 2, pair, 0)
    wait_out(GROUPS - 1, 1)


def _tc_body(x_ref, o_ref):
    x = x_ref[...]
    rows, d = x.shape
    k = max(1, int(d * K_FRAC))
    u = pltpu.bitcast(x, jnp.uint32)
    neg = (u >> 31) == 1
    u = jnp.where(neg, ~u, u | jnp.uint32(0x80000000))
    prefix = jnp.zeros((rows, 1), jnp.uint32)
    for bit in range(31, 15, -1):
        cand = prefix | jnp.uint32(1 << bit)
        cnt = jnp.sum((u >= cand).astype(jnp.int32), axis=1, keepdims=True)
        prefix = jnp.where(cnt >= k, cand, prefix)
    tneg = (prefix >> 31) == 0
    tbits = jnp.where(tneg, ~prefix, prefix & jnp.uint32(0x7FFFFFFF))
    t = pltpu.bitcast(tbits, jnp.float32)
    o_ref[...] = x * (1.0 / (1.0 + jnp.exp(t - x)))


def _tc_call(xf):
    rows_blk = 512
    rows = ROWS - ROWS_SC
    off = ROWS_SC // rows_blk
    return pl.pallas_call(
        _tc_body,
        grid=(rows // rows_blk,),
        in_specs=[pl.BlockSpec((rows_blk, D), lambda i: (i + off, 0))],
        out_specs=pl.BlockSpec((rows_blk, D), lambda i: (i + off, 0)),
        out_shape=jax.ShapeDtypeStruct((ROWS, D), jnp.float32),
    )(xf)


def kernel(x):
    b, s, d = x.shape
    xf = x.reshape(b * s, d)
    mesh = plsc.VectorSubcoreMesh(
        core_axis_name="c", subcore_axis_name="s",
        num_cores=NC, num_subcores=NS)
    fn = pl.kernel(
        _sc_body,
        out_type=jax.ShapeDtypeStruct((ROWS_SC, D), jnp.float32),
        mesh=mesh,
        compiler_params=pltpu.CompilerParams(needs_layout_passes=False),
        scratch_types=[
            pltpu.VMEM((L, D), jnp.float32),
            pltpu.VMEM((L, D), jnp.float32),
            pltpu.VMEM((512 * L,), jnp.int32),
            pltpu.SemaphoreType.DMA,
            pltpu.SemaphoreType.DMA,
            pltpu.SemaphoreType.DMA,
            pltpu.SemaphoreType.DMA,
        ],
    )
    sc_out = fn(xf)
    tc_out = _tc_call(xf)
    out = lax.dynamic_update_slice(tc_out, sc_out, (0, 0))
    return out.reshape(b, s, d)


# speedup vs baseline: 1.4408x; 1.0269x over previous
import jax
import jax.numpy as jnp
from jax import lax
from jax.experimental import pallas as pl
from jax.experimental.pallas import tpu as pltpu
from jax.experimental.pallas import tpu_sc as plsc

K_FRAC = 0.1

NC, NS, L = 2, 16, 16
NW = NC * NS

ROWS = 16384
D = 2048
ROWS_SC = 6144
ROWS_PER_W = ROWS_SC // NW
GROUPS = ROWS_PER_W // L

IMIN = -2**31


def _sc_body(x_hbm, o_hbm, xb0, xb1, hist, isem0, isem1, osem0, osem1):
    k = max(1, int(D * K_FRAC))
    wid = lax.axis_index("s") * NC + lax.axis_index("c")
    lane = lax.iota(jnp.int32, L)
    ones = jnp.ones((L,), jnp.int32)
    zeros = jnp.zeros((L,), jnp.int32)
    kv = jnp.full((L,), k, jnp.int32)
    lane_base = lane * D
    row0 = wid * ROWS_PER_W

    bufs = (xb0, xb1)
    isems = (isem0, isem1)
    osems = (osem0, osem1)

    def start_in(g, b):
        return pltpu.async_copy(
            x_hbm.at[pl.ds((row0 + g * L), L), :], bufs[b], isems[b])

    def start_out(g, b):
        return pltpu.async_copy(
            bufs[b], o_hbm.at[pl.ds((row0 + g * L), L), :], osems[b])

    @plsc.parallel_loop(0, 256, unroll=8)
    def _z(i):
        hist[pl.ds(i * L, L)] = zeros

    def select(kcur):
        @plsc.parallel_loop(0, 256, unroll=8, carry=(zeros, zeros, zeros))
        def res(i, carry):
            acc, chosen, cntgt = carry
            d = 255 - i
            h = hist[pl.ds(d * L, L)]
            hist[pl.ds(d * L, L)] = zeros
            acc2 = acc + h
            newly = (acc < kcur) & (acc2 >= kcur)
            dv = jnp.full((L,), d, jnp.int32)
            chosen = jnp.where(newly, dv, chosen)
            cntgt = jnp.where(newly, acc, cntgt)
            return acc2, chosen, cntgt
        acc, chosen, cntgt = res
        return chosen, kcur - cntgt

    def compute(g, b, mid):
        xbuf = bufs[b]

        @plsc.parallel_loop(0, D, unroll=8)
        def _p1(j):
            col = (j + lane) & (D - 1)
            bits = plsc.bitcast(
                plsc.load_gather(xbuf, [lane, col]), jnp.int32)
            u = bits ^ (lax.shift_right_arithmetic(bits, 31) | IMIN)
            plsc.store_scatter(xbuf, [lane, col],
                               plsc.bitcast(u, jnp.float32))
            addr = (lax.shift_right_logical(u, 20) & 0xFF0) | lane
            plsc.addupdate_scatter(hist, [addr], ones)

        mid()

        d1, k2 = select(kv)

        @plsc.parallel_loop(0, D, unroll=8)
        def _p2(j):
            col = (j + lane) & (D - 1)
            u = plsc.bitcast(plsc.load_gather(xbuf, [lane, col]), jnp.int32)
            match = lax.shift_right_logical(u, 24) == d1
            addr = (lax.shift_right_logical(u, 12) & 0xFF0) | lane
            plsc.addupdate_scatter(hist, [addr], ones, mask=match)

        d2, _ = select(k2)

        t_u = ((d1 << 8) | d2) << 16
        tbits = t_u ^ (~lax.shift_right_arithmetic(t_u, 31) | IMIN)
        t = plsc.bitcast(tbits, jnp.float32)

        @plsc.parallel_loop(0, D, unroll=8)
        def _ap(j):
            col = (j + lane) & (D - 1)
            u = plsc.bitcast(plsc.load_gather(xbuf, [lane, col]), jnp.int32)
            xb = u ^ (~lax.shift_right_arithmetic(u, 31) | IMIN)
            v = plsc.bitcast(xb, jnp.float32)
            y = v / (1.0 + jnp.exp(t - v))
            plsc.store_scatter(xbuf, [lane, col], y)

    def wait_in(g, b):
        pltpu.make_async_copy(
            x_hbm.at[pl.ds((row0 + g * L), L), :], bufs[b],
            isems[b]).wait()

    def wait_out(g, b):
        pltpu.make_async_copy(
            bufs[b], o_hbm.at[pl.ds((row0 + g * L), L), :],
            osems[b]).wait()

    start_in(0, 0)
    start_in(1, 1)

    def pair(h, c):
        wait_in(2 * h, 0)

        def mid0():
            @pl.when(h > 0)
            def _():
                wait_out(2 * h - 1, 1)
                start_in(2 * h + 1, 1)

        compute(2 * h, 0, mid0)
        start_out(2 * h, 0)

        def mid1():
            @pl.when(h + 1 < GROUPS // 2)
            def _():
                wait_out(2 * h, 0)
                start_in(2 * h + 2, 0)

        wait_in(2 * h + 1, 1)
        compute(2 * h + 1, 1, mid1)
        start_out(2 * h + 1, 1)

        @pl.when(h + 1 >= GROUPS // 2)
        def _():
            wait_out(2 * h, 0)
        return c

    lax.fori_loop(0, GROUPS // 2, pair, 0)
    wait_out(GROUPS - 1, 1)


def _tc_body(x_ref, o_ref):
    x = x_ref[...]
    rows, d = x.shape
    k = max(1, int(d * K_FRAC))
    u = pltpu.bitcast(x, jnp.uint32)
    neg = (u >> 31) == 1
    u = jnp.where(neg, ~u, u | jnp.uint32(0x80000000))
    prefix = jnp.zeros((rows, 1), jnp.uint32)
    for bit in range(31, 15, -1):
        cand = prefix | jnp.uint32(1 << bit)
        cnt = jnp.sum((u >= cand).astype(jnp.int32), axis=1, keepdims=True)
        prefix = jnp.where(cnt >= k, cand, prefix)
    tneg = (prefix >> 31) == 0
    tbits = jnp.where(tneg, ~prefix, prefix & jnp.uint32(0x7FFFFFFF))
    t = pltpu.bitcast(tbits, jnp.float32)
    o_ref[...] = x * (1.0 / (1.0 + jnp.exp(t - x)))


def _tc_call(xf):
    rows_blk = 512
    rows = ROWS - ROWS_SC
    off = ROWS_SC // rows_blk
    return pl.pallas_call(
        _tc_body,
        grid=(rows // rows_blk,),
        in_specs=[pl.BlockSpec((rows_blk, D), lambda i: (i + off, 0))],
        out_specs=pl.BlockSpec((rows_blk, D), lambda i: (i + off, 0)),
        out_shape=jax.ShapeDtypeStruct((ROWS, D), jnp.float32),
    )(xf)


def kernel(x):
    b, s, d = x.shape
    xf = x.reshape(b * s, d)
    mesh = plsc.VectorSubcoreMesh(
        core_axis_name="c", subcore_axis_name="s",
        num_cores=NC, num_subcores=NS)
    fn = pl.kernel(
        _sc_body,
        out_type=jax.ShapeDtypeStruct((ROWS_SC, D), jnp.float32),
        mesh=mesh,
        compiler_params=pltpu.CompilerParams(needs_layout_passes=False),
        scratch_types=[
            pltpu.VMEM((L, D), jnp.float32),
            pltpu.VMEM((L, D), jnp.float32),
            pltpu.VMEM((256 * L,), jnp.int32),
            pltpu.SemaphoreType.DMA,
            pltpu.SemaphoreType.DMA,
            pltpu.SemaphoreType.DMA,
            pltpu.SemaphoreType.DMA,
        ],
    )
    sc_out = fn(xf)
    tc_out = _tc_call(xf)
    out = lax.dynamic_update_slice(tc_out, sc_out, (0, 0))
    return out.reshape(b, s, d)


# speedup vs baseline: 1.4411x; 1.0002x over previous
import jax
import jax.numpy as jnp
from jax import lax
from jax.experimental import pallas as pl
from jax.experimental.pallas import tpu as pltpu
from jax.experimental.pallas import tpu_sc as plsc

K_FRAC = 0.1

NC, NS, L = 2, 16, 16
NW = NC * NS

ROWS = 16384
D = 2048
ROWS_SC = 6144
ROWS_PER_W = ROWS_SC // NW
GROUPS = ROWS_PER_W // L

IMIN = -2**31


def _sc_body(x_hbm, o_hbm, xb0, xb1, hist, isem0, isem1, osem0, osem1):
    k = max(1, int(D * K_FRAC))
    wid = lax.axis_index("s") * NC + lax.axis_index("c")
    lane = lax.iota(jnp.int32, L)
    ones = jnp.ones((L,), jnp.int32)
    zeros = jnp.zeros((L,), jnp.int32)
    kv = jnp.full((L,), k, jnp.int32)
    row0 = wid * ROWS_PER_W

    bufs = (xb0, xb1)
    isems = (isem0, isem1)
    osems = (osem0, osem1)

    def start_in(g, b):
        return pltpu.async_copy(
            x_hbm.at[pl.ds((row0 + g * L), L), :], bufs[b], isems[b])

    def start_out(g, b):
        return pltpu.async_copy(
            bufs[b], o_hbm.at[pl.ds((row0 + g * L), L), :], osems[b])

    @plsc.parallel_loop(0, 256, unroll=8)
    def _z(i):
        hist[pl.ds(i * L, L)] = zeros

    def select(kcur):
        @plsc.parallel_loop(0, 256, unroll=8, carry=(zeros, zeros, zeros))
        def res(i, carry):
            acc, chosen, cntgt = carry
            d = 255 - i
            h = hist[pl.ds(d * L, L)]
            hist[pl.ds(d * L, L)] = zeros
            acc2 = acc + h
            newly = (acc < kcur) & (acc2 >= kcur)
            dv = jnp.full((L,), d, jnp.int32)
            chosen = jnp.where(newly, dv, chosen)
            cntgt = jnp.where(newly, acc, cntgt)
            return acc2, chosen, cntgt
        acc, chosen, cntgt = res
        return chosen, kcur - cntgt

    def compute(g, b, mid):
        xbuf = bufs[b]

        @plsc.parallel_loop(0, D, unroll=8)
        def _p1(j):
            col = (j + lane) & (D - 1)
            bits = plsc.bitcast(
                plsc.load_gather(xbuf, [lane, col]), jnp.int32)
            u = bits ^ (lax.shift_right_arithmetic(bits, 31) | IMIN)
            plsc.store_scatter(xbuf, [lane, col],
                               plsc.bitcast(u, jnp.float32))
            addr = (lax.shift_right_logical(u, 20) & 0xFF0) | lane
            plsc.addupdate_scatter(hist, [addr], ones)

        mid()

        d1, k2 = select(kv)

        @plsc.parallel_loop(0, D, unroll=8)
        def _p2(j):
            col = (j + lane) & (D - 1)
            u = plsc.bitcast(plsc.load_gather(xbuf, [lane, col]), jnp.int32)
            match = lax.shift_right_logical(u, 24) == d1
            addr = (lax.shift_right_logical(u, 12) & 0xFF0) | lane
            plsc.addupdate_scatter(hist, [addr], ones, mask=match)

        d2, _ = select(k2)

        t_u = ((d1 << 8) | d2) << 16
        tbits = t_u ^ (~lax.shift_right_arithmetic(t_u, 31) | IMIN)
        t = plsc.bitcast(tbits, jnp.float32)

        @plsc.parallel_loop(0, D, unroll=8)
        def _ap(j):
            col = (j + lane) & (D - 1)
            u = plsc.bitcast(plsc.load_gather(xbuf, [lane, col]), jnp.int32)
            xb = u ^ (~lax.shift_right_arithmetic(u, 31) | IMIN)
            v = plsc.bitcast(xb, jnp.float32)
            y = v / (1.0 + jnp.exp(t - v))
            plsc.store_scatter(xbuf, [lane, col], y)

    def wait_in(g, b):
        pltpu.make_async_copy(
            x_hbm.at[pl.ds((row0 + g * L), L), :], bufs[b],
            isems[b]).wait()

    def wait_out(g, b):
        pltpu.make_async_copy(
            bufs[b], o_hbm.at[pl.ds((row0 + g * L), L), :],
            osems[b]).wait()

    start_in(0, 0)
    start_in(1, 1)

    def pair(h, c):
        wait_in(2 * h, 0)

        def mid0():
            @pl.when(h > 0)
            def _():
                wait_out(2 * h - 1, 1)
                start_in(2 * h + 1, 1)

        compute(2 * h, 0, mid0)
        start_out(2 * h, 0)

        def mid1():
            @pl.when(h + 1 < GROUPS // 2)
            def _():
                wait_out(2 * h, 0)
                start_in(2 * h + 2, 0)

        wait_in(2 * h + 1, 1)
        compute(2 * h + 1, 1, mid1)
        start_out(2 * h + 1, 1)

        @pl.when(h + 1 >= GROUPS // 2)
        def _():
            wait_out(2 * h, 0)
        return c

    lax.fori_loop(0, GROUPS // 2, pair, 0)
    wait_out(GROUPS - 1, 1)


def _tc_body(x_ref, o_ref):
    x = x_ref[...]
    rows, d = x.shape
    k = max(1, int(d * K_FRAC))
    u = pltpu.bitcast(x, jnp.uint32)
    neg = (u >> 31) == 1
    u = jnp.where(neg, ~u, u | jnp.uint32(0x80000000))
    prefix = jnp.zeros((rows, 1), jnp.uint32)
    for bit in range(31, 15, -1):
        cand = prefix | jnp.uint32(1 << bit)
        cnt = jnp.sum((u >= cand).astype(jnp.int32), axis=1, keepdims=True)
        prefix = jnp.where(cnt >= k, cand, prefix)
    tneg = (prefix >> 31) == 0
    tbits = jnp.where(tneg, ~prefix, prefix & jnp.uint32(0x7FFFFFFF))
    t = pltpu.bitcast(tbits, jnp.float32)
    o_ref[...] = x * (1.0 / (1.0 + jnp.exp(t - x)))


def _tc_call(xf):
    rows_blk = 512
    rows = ROWS - ROWS_SC
    off = ROWS_SC // rows_blk
    return pl.pallas_call(
        _tc_body,
        grid=(rows // rows_blk,),
        in_specs=[pl.BlockSpec((rows_blk, D), lambda i: (i + off, 0))],
        out_specs=pl.BlockSpec((rows_blk, D), lambda i: (i + off, 0)),
        out_shape=jax.ShapeDtypeStruct((ROWS, D), jnp.float32),
    )(xf)


def kernel(x):
    b, s, d = x.shape
    xf = x.reshape(b * s, d)
    mesh = plsc.VectorSubcoreMesh(
        core_axis_name="c", subcore_axis_name="s",
        num_cores=NC, num_subcores=NS)
    fn = pl.kernel(
        _sc_body,
        out_type=jax.ShapeDtypeStruct((ROWS_SC, D), jnp.float32),
        mesh=mesh,
        compiler_params=pltpu.CompilerParams(needs_layout_passes=False),
        scratch_types=[
            pltpu.VMEM((L, D), jnp.float32),
            pltpu.VMEM((L, D), jnp.float32),
            pltpu.VMEM((256 * L,), jnp.int32),
            pltpu.SemaphoreType.DMA,
            pltpu.SemaphoreType.DMA,
            pltpu.SemaphoreType.DMA,
            pltpu.SemaphoreType.DMA,
        ],
    )
    sc_out = fn(xf)
    tc_out = _tc_call(xf)
    out = lax.dynamic_update_slice(tc_out, sc_out, (0, 0))
    return out.reshape(b, s, d)
